# baseline (speedup 1.0000x reference)
import functools

import jax
import jax.numpy as jnp
from jax import lax
from jax.experimental import pallas as pl
from jax.experimental.pallas import tpu as pltpu
from jax.experimental.pallas import tpu_sc as plsc

N = 10000
E = 320000
D_IN = 128
D_HID = 128
N_CLASSES = 64

NUM_SC = 2
NUM_SUBCORES = 16
NUM_TILES = NUM_SC * NUM_SUBCORES

CHUNK = 128
CHUNKS = 80
E_PER_TILE = CHUNKS * CHUNK
E_PAD = NUM_TILES * E_PER_TILE
N_PAD = 10240
ROWS_PER_SUBCORE = N_PAD // NUM_SUBCORES
DEG_W = 16


def _make_sc_scatter(width, with_deg):
  mesh = plsc.VectorSubcoreMesh(core_axis_name="c", subcore_axis_name="s")

  out_type = [jax.ShapeDtypeStruct((NUM_SC, N_PAD, width), jnp.float32)]
  scratch = [
      pltpu.VMEM((2, CHUNK), jnp.int32),
      pltpu.VMEM((2, CHUNK), jnp.int32),
      pltpu.VMEM((CHUNK, width), jnp.float32),
      pltpu.VMEM((CHUNK, width), jnp.float32),
      pltpu.VMEM_SHARED((N_PAD, width), jnp.float32),
      pltpu.SemaphoreType.DMA,
      pltpu.SemaphoreType.DMA,
      pltpu.SemaphoreType.DMA,
      pltpu.SemaphoreType.DMA,
      pltpu.SemaphoreType.DMA,
      pltpu.SemaphoreType.DMA,
      pltpu.SemaphoreType.DMA,
  ]
  if with_deg:
    out_type.append(jax.ShapeDtypeStruct((NUM_SC, N_PAD, DEG_W), jnp.float32))
    scratch += [
        pltpu.VMEM((CHUNK, DEG_W), jnp.float32),
        pltpu.VMEM_SHARED((N_PAD, DEG_W), jnp.float32),
    ]

  def sc_kernel(*refs):
    if with_deg:
      (t_hbm, e_hbm, zero_hbm, zdeg_hbm, ones_hbm,
       out_hbm, deg_hbm,
       sring, dring, rows0, rows1, acc,
       sem0, sem1, ssem0, ssem1, dsem0, dsem1, msem,
       ones_v, dacc) = refs
    else:
      (t_hbm, e_hbm, zero_hbm,
       out_hbm,
       sring, dring, rows0, rows1, acc,
       sem0, sem1, ssem0, ssem1, dsem0, dsem1, msem) = refs
    c = lax.axis_index("c")
    s = lax.axis_index("s")
    w = c * NUM_SUBCORES + s

    row0 = s * ROWS_PER_SUBCORE
    pltpu.sync_copy(zero_hbm, acc.at[pl.ds(row0, ROWS_PER_SUBCORE)])
    if with_deg:
      pltpu.sync_copy(zdeg_hbm, dacc.at[pl.ds(row0, ROWS_PER_SUBCORE)])
      pltpu.sync_copy(ones_hbm, ones_v)
    plsc.subcore_barrier()

    def fetch_idx(ring, which, i, slot, sem):
      pltpu.async_copy(e_hbm.at[which, w, i], ring.at[slot], sem)

    def wait_idx(ring, slot, sem):
      pltpu.make_async_copy(e_hbm.at[0, w, 0], ring.at[slot], sem).wait()

    def do_scatter(buf, slot):
      cp = pltpu.async_copy(buf, acc.at[dring.at[slot]], msem, add=True)
      if with_deg:
        pltpu.sync_copy(ones_v, dacc.at[dring.at[slot]], add=True)
      cp.wait()

    fetch_idx(sring, 0, 0, 0, ssem0)
    fetch_idx(sring, 0, 1, 1, ssem1)
    fetch_idx(dring, 1, 0, 0, dsem0)
    fetch_idx(dring, 1, 1, 1, dsem1)
    wait_idx(sring, 0, ssem0)
    pltpu.async_copy(t_hbm.at[sring.at[0]], rows0, sem0)
    wait_idx(sring, 1, ssem1)
    pltpu.async_copy(t_hbm.at[sring.at[1]], rows1, sem1)

    @pl.loop(0, CHUNKS - 2, step=2)
    def _(i):
      pltpu.make_async_copy(t_hbm.at[sring.at[0]], rows0, sem0).wait()
      fetch_idx(sring, 0, i + 2, 0, ssem0)
      wait_idx(dring, 0, dsem0)
      do_scatter(rows0, 0)
      fetch_idx(dring, 1, i + 2, 0, dsem0)
      wait_idx(sring, 0, ssem0)
      pltpu.async_copy(t_hbm.at[sring.at[0]], rows0, sem0)
      pltpu.make_async_copy(t_hbm.at[sring.at[1]], rows1, sem1).wait()
      fetch_idx(sring, 0, i + 3, 1, ssem1)
      wait_idx(dring, 1, dsem1)
      do_scatter(rows1, 1)
      fetch_idx(dring, 1, i + 3, 1, dsem1)
      wait_idx(sring, 1, ssem1)
      pltpu.async_copy(t_hbm.at[sring.at[1]], rows1, sem1)

    pltpu.make_async_copy(t_hbm.at[sring.at[0]], rows0, sem0).wait()
    wait_idx(dring, 0, dsem0)
    do_scatter(rows0, 0)
    pltpu.make_async_copy(t_hbm.at[sring.at[1]], rows1, sem1).wait()
    wait_idx(dring, 1, dsem1)
    do_scatter(rows1, 1)

    plsc.subcore_barrier()
    pltpu.sync_copy(acc.at[pl.ds(row0, ROWS_PER_SUBCORE)],
                    out_hbm.at[c, pl.ds(row0, ROWS_PER_SUBCORE)])
    if with_deg:
      pltpu.sync_copy(dacc.at[pl.ds(row0, ROWS_PER_SUBCORE)],
                      deg_hbm.at[c, pl.ds(row0, ROWS_PER_SUBCORE)])

  return pl.kernel(
      sc_kernel,
      out_type=out_type,
      mesh=mesh,
      compiler_params=pltpu.CompilerParams(use_tc_tiling_on_sc=False),
      scratch_types=scratch,
  )


_sc_scatter_l1 = _make_sc_scatter(D_HID, with_deg=True)
_sc_scatter_l2 = _make_sc_scatter(N_CLASSES, with_deg=False)


_BLK = 1024


def _tc_matmul(x, w, b=None):
  k, m = w.shape

  def body(x_ref, w_ref, o_ref):
    o_ref[...] = jnp.dot(x_ref[...], w_ref[...],
                         preferred_element_type=jnp.float32)

  def body_b(x_ref, w_ref, b_ref, o_ref):
    o_ref[...] = jnp.dot(x_ref[...], w_ref[...],
                         preferred_element_type=jnp.float32) + b_ref[...]

  args = (x, w) if b is None else (x, w, b)
  in_specs = [
      pl.BlockSpec((_BLK, k), lambda i: (i, 0)),
      pl.BlockSpec((k, m), lambda i: (0, 0)),
  ]
  if b is not None:
    in_specs.append(pl.BlockSpec((1, m), lambda i: (0, 0)))
  return pl.pallas_call(
      body if b is None else body_b,
      grid=(N_PAD // _BLK,),
      in_specs=in_specs,
      out_specs=pl.BlockSpec((_BLK, m), lambda i: (i, 0)),
      out_shape=jax.ShapeDtypeStruct((N_PAD, m), jnp.float32),
  )(*args)


def _tc_h1(p0, p1, d0, d1, s1):
  def body(p0_ref, p1_ref, d0_ref, d1_ref, s1_ref, h_ref, r_ref):
    agg = p0_ref[...] + p1_ref[...]
    deg = (d0_ref[...] + d1_ref[...])[:, :1]
    recip = 1.0 / jnp.maximum(deg, 1.0)
    h_ref[...] = jnp.maximum(s1_ref[...] + agg * recip, 0.0)
    r_ref[...] = jnp.broadcast_to(recip, (_BLK, N_CLASSES))

  return pl.pallas_call(
      body,
      grid=(N_PAD // _BLK,),
      in_specs=[
          pl.BlockSpec((_BLK, D_HID), lambda i: (i, 0)),
          pl.BlockSpec((_BLK, D_HID), lambda i: (i, 0)),
          pl.BlockSpec((_BLK, DEG_W), lambda i: (i, 0)),
          pl.BlockSpec((_BLK, DEG_W), lambda i: (i, 0)),
          pl.BlockSpec((_BLK, D_HID), lambda i: (i, 0)),
      ],
      out_specs=[
          pl.BlockSpec((_BLK, D_HID), lambda i: (i, 0)),
          pl.BlockSpec((_BLK, N_CLASSES), lambda i: (i, 0)),
      ],
      out_shape=[
          jax.ShapeDtypeStruct((N_PAD, D_HID), jnp.float32),
          jax.ShapeDtypeStruct((N_PAD, N_CLASSES), jnp.float32),
      ],
  )(p0, p1, d0, d1, s1)


def _tc_final(q0, q1, s2, recip):
  def body(q0_ref, q1_ref, s2_ref, r_ref, o_ref):
    o_ref[...] = s2_ref[...] + (q0_ref[...] + q1_ref[...]) * r_ref[...]

  blk = 1000
  spec = pl.BlockSpec((blk, N_CLASSES), lambda i: (i, 0))
  return pl.pallas_call(
      body,
      grid=(N // blk,),
      in_specs=[spec, spec, spec, spec],
      out_specs=spec,
      out_shape=jax.ShapeDtypeStruct((N, N_CLASSES), jnp.float32),
  )(q0, q1, s2, recip)


@jax.jit
def kernel(features, edge_index, W_self1, W_neigh1, b1, W_self2, W_neigh2, b2):
  ei = edge_index.astype(jnp.int32)
  pad = N + jax.lax.iota(jnp.int32, E_PAD - E) % (N_PAD - N)
  e_view = jnp.concatenate([ei, jnp.stack([pad, pad])], axis=1).reshape(
      2, NUM_TILES, CHUNKS, CHUNK)

  x_pad = jnp.pad(features, ((0, N_PAD - N), (0, 0)))
  zeros1 = jnp.zeros((ROWS_PER_SUBCORE, D_HID), jnp.float32)
  zeros2 = jnp.zeros((ROWS_PER_SUBCORE, N_CLASSES), jnp.float32)
  zerosd = jnp.zeros((ROWS_PER_SUBCORE, DEG_W), jnp.float32)
  ones = jnp.ones((CHUNK, DEG_W), jnp.float32)

  t1 = _tc_matmul(x_pad, W_neigh1)
  p1, degp = _sc_scatter_l1(t1, e_view, zeros1, zerosd, ones)
  s1 = _tc_matmul(x_pad, W_self1, b1.reshape(1, -1))
  h1, recip = _tc_h1(p1[0], p1[1], degp[0], degp[1], s1)
  t2 = _tc_matmul(h1, W_neigh2)
  (p2,) = _sc_scatter_l2(t2, e_view, zeros2)
  s2 = _tc_matmul(h1, W_self2, b2.reshape(1, -1))
  return _tc_final(p2[0], p2[1], s2, recip)


# speedup vs baseline: 1.0867x; 1.0867x over previous
import functools

import jax
import jax.numpy as jnp
from jax import lax
from jax.experimental import pallas as pl
from jax.experimental.pallas import tpu as pltpu
from jax.experimental.pallas import tpu_sc as plsc

N = 10000
E = 320000
D_IN = 128
D_HID = 128
N_CLASSES = 64

NUM_SC = 2
NUM_SUBCORES = 16
NUM_TILES = NUM_SC * NUM_SUBCORES

CHUNK = 128
E_PER_TILE = E // NUM_TILES
MAIN_CHUNKS = E_PER_TILE // CHUNK
TAIL = E_PER_TILE - MAIN_CHUNKS * CHUNK
N_PAD = 10240
ROWS_PER_SUBCORE = N_PAD // NUM_SUBCORES
DEG_W = 16


def _make_sc_scatter(width, with_deg):
  mesh = plsc.VectorSubcoreMesh(core_axis_name="c", subcore_axis_name="s")

  out_type = [jax.ShapeDtypeStruct((NUM_SC, N_PAD, width), jnp.float32)]
  scratch = [
      pltpu.VMEM((2, CHUNK), jnp.int32),
      pltpu.VMEM((2, CHUNK), jnp.int32),
      pltpu.VMEM((2, TAIL), jnp.int32),
      pltpu.VMEM((CHUNK, width), jnp.float32),
      pltpu.VMEM((CHUNK, width), jnp.float32),
      pltpu.VMEM((TAIL, width), jnp.float32),
      pltpu.VMEM_SHARED((N_PAD, width), jnp.float32),
      pltpu.SemaphoreType.DMA,
      pltpu.SemaphoreType.DMA,
      pltpu.SemaphoreType.DMA,
      pltpu.SemaphoreType.DMA,
      pltpu.SemaphoreType.DMA,
      pltpu.SemaphoreType.DMA,
      pltpu.SemaphoreType.DMA,
  ]
  if with_deg:
    out_type.append(jax.ShapeDtypeStruct((NUM_SC, N_PAD, DEG_W), jnp.float32))
    scratch += [
        pltpu.VMEM((CHUNK, DEG_W), jnp.float32),
        pltpu.VMEM_SHARED((N_PAD, DEG_W), jnp.float32),
    ]

  def sc_kernel(*refs):
    if with_deg:
      (t_hbm, e_hbm,
       out_hbm, deg_hbm,
       sring, dring, tring, rows0, rows1, trows, acc,
       sem0, sem1, ssem0, ssem1, dsem0, dsem1, msem,
       ones_v, dacc) = refs
    else:
      (t_hbm, e_hbm,
       out_hbm,
       sring, dring, tring, rows0, rows1, trows, acc,
       sem0, sem1, ssem0, ssem1, dsem0, dsem1, msem) = refs
    c = lax.axis_index("c")
    s = lax.axis_index("s")
    w = c * NUM_SUBCORES + s
    base = w * E_PER_TILE

    row0 = s * ROWS_PER_SUBCORE
    z16 = jnp.zeros((16,), jnp.float32)

    @pl.loop(0, CHUNK)
    def _(r):
      for j in range(width // 16):
        rows0[r, pl.ds(j * 16, 16)] = z16

    for k in range(ROWS_PER_SUBCORE // CHUNK):
      pltpu.sync_copy(rows0, acc.at[pl.ds(row0 + k * CHUNK, CHUNK)])
    if with_deg:
      @pl.loop(0, CHUNK)
      def _(r):
        ones_v[r, pl.ds(0, DEG_W)] = z16
      for k in range(ROWS_PER_SUBCORE // CHUNK):
        pltpu.sync_copy(ones_v, dacc.at[pl.ds(row0 + k * CHUNK, CHUNK)])
      one16 = jnp.ones((16,), jnp.float32)

      @pl.loop(0, CHUNK)
      def _(r):
        ones_v[r, pl.ds(0, DEG_W)] = one16
    plsc.subcore_barrier()

    def fetch_idx(ring, which, i, slot, sem):
      pltpu.async_copy(e_hbm.at[which, pl.ds(base + i * CHUNK, CHUNK)],
                       ring.at[slot], sem)

    def wait_idx(ring, slot, sem):
      pltpu.make_async_copy(e_hbm.at[0, pl.ds(0, CHUNK)],
                            ring.at[slot], sem).wait()

    def do_scatter(buf, slot):
      cp = pltpu.async_copy(buf, acc.at[dring.at[slot]], msem, add=True)
      if with_deg:
        pltpu.sync_copy(ones_v, dacc.at[dring.at[slot]], add=True)
      cp.wait()

    fetch_idx(sring, 0, 0, 0, ssem0)
    fetch_idx(sring, 0, 1, 1, ssem1)
    fetch_idx(dring, 1, 0, 0, dsem0)
    fetch_idx(dring, 1, 1, 1, dsem1)
    wait_idx(sring, 0, ssem0)
    pltpu.async_copy(t_hbm.at[sring.at[0]], rows0, sem0)
    wait_idx(sring, 1, ssem1)
    pltpu.async_copy(t_hbm.at[sring.at[1]], rows1, sem1)

    @pl.loop(0, MAIN_CHUNKS - 2, step=2)
    def _(i):
      pltpu.make_async_copy(t_hbm.at[sring.at[0]], rows0, sem0).wait()
      fetch_idx(sring, 0, i + 2, 0, ssem0)
      wait_idx(dring, 0, dsem0)
      do_scatter(rows0, 0)
      fetch_idx(dring, 1, i + 2, 0, dsem0)
      wait_idx(sring, 0, ssem0)
      pltpu.async_copy(t_hbm.at[sring.at[0]], rows0, sem0)
      pltpu.make_async_copy(t_hbm.at[sring.at[1]], rows1, sem1).wait()
      fetch_idx(sring, 0, i + 3, 1, ssem1)
      wait_idx(dring, 1, dsem1)
      do_scatter(rows1, 1)
      fetch_idx(dring, 1, i + 3, 1, dsem1)
      wait_idx(sring, 1, ssem1)
      pltpu.async_copy(t_hbm.at[sring.at[1]], rows1, sem1)

    pltpu.make_async_copy(t_hbm.at[sring.at[0]], rows0, sem0).wait()
    wait_idx(dring, 0, dsem0)
    do_scatter(rows0, 0)
    pltpu.async_copy(e_hbm.at[0, pl.ds(base + MAIN_CHUNKS * CHUNK, TAIL)],
                     tring.at[0], ssem0)
    pltpu.async_copy(e_hbm.at[1, pl.ds(base + MAIN_CHUNKS * CHUNK, TAIL)],
                     tring.at[1], dsem0)
    pltpu.make_async_copy(t_hbm.at[sring.at[1]], rows1, sem1).wait()
    wait_idx(dring, 1, dsem1)
    do_scatter(rows1, 1)
    pltpu.make_async_copy(e_hbm.at[0, pl.ds(0, TAIL)], tring.at[0],
                          ssem0).wait()
    pltpu.sync_copy(t_hbm.at[tring.at[0]], trows)
    pltpu.make_async_copy(e_hbm.at[0, pl.ds(0, TAIL)], tring.at[1],
                          dsem0).wait()
    pltpu.sync_copy(trows, acc.at[tring.at[1]], add=True)
    if with_deg:
      pltpu.sync_copy(ones_v.at[pl.ds(0, TAIL)], dacc.at[tring.at[1]],
                      add=True)

    plsc.subcore_barrier()
    pltpu.sync_copy(acc.at[pl.ds(row0, ROWS_PER_SUBCORE)],
                    out_hbm.at[c, pl.ds(row0, ROWS_PER_SUBCORE)])
    if with_deg:
      pltpu.sync_copy(dacc.at[pl.ds(row0, ROWS_PER_SUBCORE)],
                      deg_hbm.at[c, pl.ds(row0, ROWS_PER_SUBCORE)])

  return pl.kernel(
      sc_kernel,
      out_type=out_type,
      mesh=mesh,
      compiler_params=pltpu.CompilerParams(use_tc_tiling_on_sc=False),
      scratch_types=scratch,
  )


_sc_scatter_l1 = _make_sc_scatter(D_HID, with_deg=True)
_sc_scatter_l2 = _make_sc_scatter(N_CLASSES, with_deg=False)


_BLK = 1024


def _tc_layer1(x_pad, w_neigh, w_self, b):
  def body(x_ref, wn_ref, ws_ref, b_ref, t_ref, s_ref):
    x = x_ref[...]
    t_ref[...] = jnp.dot(x, wn_ref[...], preferred_element_type=jnp.float32)
    s_ref[...] = jnp.dot(x, ws_ref[...], preferred_element_type=jnp.float32) + b_ref[...]

  return pl.pallas_call(
      body,
      grid=(N_PAD // _BLK,),
      in_specs=[
          pl.BlockSpec((_BLK, D_IN), lambda i: (i, 0)),
          pl.BlockSpec((D_IN, D_HID), lambda i: (0, 0)),
          pl.BlockSpec((D_IN, D_HID), lambda i: (0, 0)),
          pl.BlockSpec((1, D_HID), lambda i: (0, 0)),
      ],
      out_specs=[
          pl.BlockSpec((_BLK, D_HID), lambda i: (i, 0)),
          pl.BlockSpec((_BLK, D_HID), lambda i: (i, 0)),
      ],
      out_shape=[
          jax.ShapeDtypeStruct((N_PAD, D_HID), jnp.float32),
          jax.ShapeDtypeStruct((N_PAD, D_HID), jnp.float32),
      ],
  )(x_pad, w_neigh, w_self, b)


def _tc_mid(p0, p1, d0, d1, s1, w_neigh, w_self, b):
  def body(p0_ref, p1_ref, d0_ref, d1_ref, s1_ref, wn_ref, ws_ref, b_ref,
           t2_ref, s2_ref, r_ref):
    agg = p0_ref[...] + p1_ref[...]
    deg = (d0_ref[...] + d1_ref[...])[:, :1]
    recip = 1.0 / jnp.maximum(deg, 1.0)
    h1 = jnp.maximum(s1_ref[...] + agg * recip, 0.0)
    t2_ref[...] = jnp.dot(h1, wn_ref[...], preferred_element_type=jnp.float32)
    s2_ref[...] = (jnp.dot(h1, ws_ref[...], preferred_element_type=jnp.float32)
                   + b_ref[...])
    r_ref[...] = jnp.broadcast_to(recip, (_BLK, DEG_W))

  return pl.pallas_call(
      body,
      grid=(N_PAD // _BLK,),
      in_specs=[
          pl.BlockSpec((_BLK, D_HID), lambda i: (i, 0)),
          pl.BlockSpec((_BLK, D_HID), lambda i: (i, 0)),
          pl.BlockSpec((_BLK, DEG_W), lambda i: (i, 0)),
          pl.BlockSpec((_BLK, DEG_W), lambda i: (i, 0)),
          pl.BlockSpec((_BLK, D_HID), lambda i: (i, 0)),
          pl.BlockSpec((D_HID, N_CLASSES), lambda i: (0, 0)),
          pl.BlockSpec((D_HID, N_CLASSES), lambda i: (0, 0)),
          pl.BlockSpec((1, N_CLASSES), lambda i: (0, 0)),
      ],
      out_specs=[
          pl.BlockSpec((_BLK, N_CLASSES), lambda i: (i, 0)),
          pl.BlockSpec((_BLK, N_CLASSES), lambda i: (i, 0)),
          pl.BlockSpec((_BLK, DEG_W), lambda i: (i, 0)),
      ],
      out_shape=[
          jax.ShapeDtypeStruct((N_PAD, N_CLASSES), jnp.float32),
          jax.ShapeDtypeStruct((N_PAD, N_CLASSES), jnp.float32),
          jax.ShapeDtypeStruct((N_PAD, DEG_W), jnp.float32),
      ],
  )(p0, p1, d0, d1, s1, w_neigh, w_self, b)


def _tc_final(q0, q1, s2, recip):
  def body(q0_ref, q1_ref, s2_ref, r_ref, o_ref):
    r = jnp.broadcast_to(r_ref[...][:, :1], (1000, N_CLASSES))
    o_ref[...] = s2_ref[...] + (q0_ref[...] + q1_ref[...]) * r

  blk = 1000
  spec = pl.BlockSpec((blk, N_CLASSES), lambda i: (i, 0))
  rspec = pl.BlockSpec((blk, DEG_W), lambda i: (i, 0))
  return pl.pallas_call(
      body,
      grid=(N // blk,),
      in_specs=[spec, spec, spec, rspec],
      out_specs=spec,
      out_shape=jax.ShapeDtypeStruct((N, N_CLASSES), jnp.float32),
  )(q0, q1, s2, recip)


@jax.jit
def kernel(features, edge_index, W_self1, W_neigh1, b1, W_self2, W_neigh2, b2):
  ei = edge_index.astype(jnp.int32)
  x_pad = jnp.pad(features, ((0, N_PAD - N), (0, 0)))

  t1, s1 = _tc_layer1(x_pad, W_neigh1, W_self1, b1.reshape(1, -1))
  p1, degp = _sc_scatter_l1(t1, ei)
  t2, s2, recip = _tc_mid(p1[0], p1[1], degp[0], degp[1], s1,
                          W_neigh2, W_self2, b2.reshape(1, -1))
  (p2,) = _sc_scatter_l2(t2, ei)
  return _tc_final(p2[0], p2[1], s2, recip)


# speedup vs baseline: 1.0919x; 1.0048x over previous
import functools

import jax
import jax.numpy as jnp
from jax import lax
from jax.experimental import pallas as pl
from jax.experimental.pallas import tpu as pltpu
from jax.experimental.pallas import tpu_sc as plsc

N = 10000
E = 320000
D_IN = 128
D_HID = 128
N_CLASSES = 64

NUM_SC = 2
NUM_SUBCORES = 16
NUM_TILES = NUM_SC * NUM_SUBCORES

CHUNK = 128
E_PER_TILE = E // NUM_TILES
MAIN_CHUNKS = E_PER_TILE // CHUNK
TAIL = E_PER_TILE - MAIN_CHUNKS * CHUNK
N_PAD = 10240
ROWS_PER_SUBCORE = N_PAD // NUM_SUBCORES
DEG_W = 16


def _make_sc_scatter(width, with_deg):
  mesh = plsc.VectorSubcoreMesh(core_axis_name="c", subcore_axis_name="s")

  out_type = [jax.ShapeDtypeStruct((NUM_SC, N_PAD, width), jnp.float32)]
  scratch = [
      pltpu.VMEM((2, CHUNK), jnp.int32),
      pltpu.VMEM((2, CHUNK), jnp.int32),
      pltpu.VMEM((2, TAIL), jnp.int32),
      pltpu.VMEM((CHUNK, width), jnp.float32),
      pltpu.VMEM((CHUNK, width), jnp.float32),
      pltpu.VMEM((TAIL, width), jnp.float32),
      pltpu.VMEM_SHARED((N_PAD, width), jnp.float32),
      pltpu.SemaphoreType.DMA,
      pltpu.SemaphoreType.DMA,
      pltpu.SemaphoreType.DMA,
      pltpu.SemaphoreType.DMA,
      pltpu.SemaphoreType.DMA,
      pltpu.SemaphoreType.DMA,
      pltpu.SemaphoreType.DMA,
  ]
  if with_deg:
    out_type.append(jax.ShapeDtypeStruct((NUM_SC, N_PAD, DEG_W), jnp.float32))
    scratch += [
        pltpu.VMEM((CHUNK, DEG_W), jnp.float32),
        pltpu.VMEM_SHARED((N_PAD, DEG_W), jnp.float32),
    ]

  def sc_kernel(*refs):
    if with_deg:
      (t_hbm, e_hbm,
       out_hbm, deg_hbm,
       sring, dring, tring, rows0, rows1, trows, acc,
       sem0, sem1, ssem0, ssem1, dsem0, dsem1, msem,
       ones_v, dacc) = refs
    else:
      (t_hbm, e_hbm,
       out_hbm,
       sring, dring, tring, rows0, rows1, trows, acc,
       sem0, sem1, ssem0, ssem1, dsem0, dsem1, msem) = refs
    c = lax.axis_index("c")
    s = lax.axis_index("s")
    w = c * NUM_SUBCORES + s
    base = w * E_PER_TILE

    row0 = s * ROWS_PER_SUBCORE
    z16 = jnp.zeros((16,), jnp.float32)

    @pl.loop(0, CHUNK)
    def _(r):
      for j in range(width // 16):
        rows0[r, pl.ds(j * 16, 16)] = z16

    for k in range(ROWS_PER_SUBCORE // CHUNK):
      pltpu.sync_copy(rows0, acc.at[pl.ds(row0 + k * CHUNK, CHUNK)])
    if with_deg:
      @pl.loop(0, CHUNK)
      def _(r):
        ones_v[r, pl.ds(0, DEG_W)] = z16
      for k in range(ROWS_PER_SUBCORE // CHUNK):
        pltpu.sync_copy(ones_v, dacc.at[pl.ds(row0 + k * CHUNK, CHUNK)])
      one16 = jnp.ones((16,), jnp.float32)

      @pl.loop(0, CHUNK)
      def _(r):
        ones_v[r, pl.ds(0, DEG_W)] = one16
    plsc.subcore_barrier()

    def fetch_idx(ring, which, i, slot, sem):
      pltpu.async_copy(e_hbm.at[which, pl.ds(base + i * CHUNK, CHUNK)],
                       ring.at[slot], sem)

    def wait_idx(ring, slot, sem):
      pltpu.make_async_copy(e_hbm.at[0, pl.ds(0, CHUNK)],
                            ring.at[slot], sem).wait()

    def do_scatter(buf, slot):
      cp = pltpu.async_copy(buf, acc.at[dring.at[slot]], msem, add=True)
      if with_deg:
        pltpu.sync_copy(ones_v, dacc.at[dring.at[slot]], add=True)
      cp.wait()

    fetch_idx(sring, 0, 0, 0, ssem0)
    fetch_idx(sring, 0, 1, 1, ssem1)
    fetch_idx(dring, 1, 0, 0, dsem0)
    fetch_idx(dring, 1, 1, 1, dsem1)
    wait_idx(sring, 0, ssem0)
    pltpu.async_copy(t_hbm.at[sring.at[0]], rows0, sem0)
    wait_idx(sring, 1, ssem1)
    pltpu.async_copy(t_hbm.at[sring.at[1]], rows1, sem1)

    @pl.loop(0, MAIN_CHUNKS - 2, step=2)
    def _(i):
      pltpu.make_async_copy(t_hbm.at[sring.at[0]], rows0, sem0).wait()
      fetch_idx(sring, 0, i + 2, 0, ssem0)
      wait_idx(dring, 0, dsem0)
      do_scatter(rows0, 0)
      fetch_idx(dring, 1, i + 2, 0, dsem0)
      wait_idx(sring, 0, ssem0)
      pltpu.async_copy(t_hbm.at[sring.at[0]], rows0, sem0)
      pltpu.make_async_copy(t_hbm.at[sring.at[1]], rows1, sem1).wait()
      fetch_idx(sring, 0, i + 3, 1, ssem1)
      wait_idx(dring, 1, dsem1)
      do_scatter(rows1, 1)
      fetch_idx(dring, 1, i + 3, 1, dsem1)
      wait_idx(sring, 1, ssem1)
      pltpu.async_copy(t_hbm.at[sring.at[1]], rows1, sem1)

    pltpu.make_async_copy(t_hbm.at[sring.at[0]], rows0, sem0).wait()
    wait_idx(dring, 0, dsem0)
    do_scatter(rows0, 0)
    pltpu.async_copy(e_hbm.at[0, pl.ds(base + MAIN_CHUNKS * CHUNK, TAIL)],
                     tring.at[0], ssem0)
    pltpu.async_copy(e_hbm.at[1, pl.ds(base + MAIN_CHUNKS * CHUNK, TAIL)],
                     tring.at[1], dsem0)
    pltpu.make_async_copy(t_hbm.at[sring.at[1]], rows1, sem1).wait()
    wait_idx(dring, 1, dsem1)
    do_scatter(rows1, 1)
    pltpu.make_async_copy(e_hbm.at[0, pl.ds(0, TAIL)], tring.at[0],
                          ssem0).wait()
    pltpu.sync_copy(t_hbm.at[tring.at[0]], trows)
    pltpu.make_async_copy(e_hbm.at[0, pl.ds(0, TAIL)], tring.at[1],
                          dsem0).wait()
    pltpu.sync_copy(trows, acc.at[tring.at[1]], add=True)
    if with_deg:
      pltpu.sync_copy(ones_v.at[pl.ds(0, TAIL)], dacc.at[tring.at[1]],
                      add=True)

    plsc.subcore_barrier()
    pltpu.sync_copy(acc.at[pl.ds(row0, ROWS_PER_SUBCORE)],
                    out_hbm.at[c, pl.ds(row0, ROWS_PER_SUBCORE)])
    if with_deg:
      pltpu.sync_copy(dacc.at[pl.ds(row0, ROWS_PER_SUBCORE)],
                      deg_hbm.at[c, pl.ds(row0, ROWS_PER_SUBCORE)])

  return pl.kernel(
      sc_kernel,
      out_type=out_type,
      mesh=mesh,
      compiler_params=pltpu.CompilerParams(use_tc_tiling_on_sc=False),
      scratch_types=scratch,
  )


_sc_scatter_l1 = _make_sc_scatter(D_HID, with_deg=True)
_sc_scatter_l2 = _make_sc_scatter(N_CLASSES, with_deg=False)


_BLK = 1000


def _tc_layer1(x, w_neigh, w_self, b):
  def body(x_ref, wn_ref, ws_ref, b_ref, t_ref, s_ref):
    x = x_ref[...]
    t_ref[...] = jnp.dot(x, wn_ref[...], preferred_element_type=jnp.float32)
    s_ref[...] = jnp.dot(x, ws_ref[...], preferred_element_type=jnp.float32) + b_ref[...]

  return pl.pallas_call(
      body,
      grid=(N // _BLK,),
      in_specs=[
          pl.BlockSpec((_BLK, D_IN), lambda i: (i, 0)),
          pl.BlockSpec((D_IN, D_HID), lambda i: (0, 0)),
          pl.BlockSpec((D_IN, D_HID), lambda i: (0, 0)),
          pl.BlockSpec((1, D_HID), lambda i: (0, 0)),
      ],
      out_specs=[
          pl.BlockSpec((_BLK, D_HID), lambda i: (i, 0)),
          pl.BlockSpec((_BLK, D_HID), lambda i: (i, 0)),
      ],
      out_shape=[
          jax.ShapeDtypeStruct((N, D_HID), jnp.float32),
          jax.ShapeDtypeStruct((N, D_HID), jnp.float32),
      ],
  )(x, w_neigh, w_self, b)


def _tc_mid(p0, p1, d0, d1, s1, w_neigh, w_self, b):
  def body(p0_ref, p1_ref, d0_ref, d1_ref, s1_ref, wn_ref, ws_ref, b_ref,
           t2_ref, s2_ref, r_ref):
    agg = p0_ref[...] + p1_ref[...]
    deg = (d0_ref[...] + d1_ref[...])[:, :1]
    recip = 1.0 / jnp.maximum(deg, 1.0)
    h1 = jnp.maximum(s1_ref[...] + agg * recip, 0.0)
    t2_ref[...] = jnp.dot(h1, wn_ref[...], preferred_element_type=jnp.float32)
    s2_ref[...] = (jnp.dot(h1, ws_ref[...], preferred_element_type=jnp.float32)
                   + b_ref[...])
    r_ref[...] = jnp.broadcast_to(recip, (_BLK, DEG_W))

  return pl.pallas_call(
      body,
      grid=(N // _BLK,),
      in_specs=[
          pl.BlockSpec((_BLK, D_HID), lambda i: (i, 0)),
          pl.BlockSpec((_BLK, D_HID), lambda i: (i, 0)),
          pl.BlockSpec((_BLK, DEG_W), lambda i: (i, 0)),
          pl.BlockSpec((_BLK, DEG_W), lambda i: (i, 0)),
          pl.BlockSpec((_BLK, D_HID), lambda i: (i, 0)),
          pl.BlockSpec((D_HID, N_CLASSES), lambda i: (0, 0)),
          pl.BlockSpec((D_HID, N_CLASSES), lambda i: (0, 0)),
          pl.BlockSpec((1, N_CLASSES), lambda i: (0, 0)),
      ],
      out_specs=[
          pl.BlockSpec((_BLK, N_CLASSES), lambda i: (i, 0)),
          pl.BlockSpec((_BLK, N_CLASSES), lambda i: (i, 0)),
          pl.BlockSpec((_BLK, DEG_W), lambda i: (i, 0)),
      ],
      out_shape=[
          jax.ShapeDtypeStruct((N, N_CLASSES), jnp.float32),
          jax.ShapeDtypeStruct((N, N_CLASSES), jnp.float32),
          jax.ShapeDtypeStruct((N, DEG_W), jnp.float32),
      ],
  )(p0, p1, d0, d1, s1, w_neigh, w_self, b)


def _tc_final(q0, q1, s2, recip):
  def body(q0_ref, q1_ref, s2_ref, r_ref, o_ref):
    r = jnp.broadcast_to(r_ref[...][:, :1], (_BLK, N_CLASSES))
    o_ref[...] = s2_ref[...] + (q0_ref[...] + q1_ref[...]) * r

  spec = pl.BlockSpec((_BLK, N_CLASSES), lambda i: (i, 0))
  rspec = pl.BlockSpec((_BLK, DEG_W), lambda i: (i, 0))
  return pl.pallas_call(
      body,
      grid=(N // _BLK,),
      in_specs=[spec, spec, spec, rspec],
      out_specs=spec,
      out_shape=jax.ShapeDtypeStruct((N, N_CLASSES), jnp.float32),
  )(q0, q1, s2, recip)


@jax.jit
def kernel(features, edge_index, W_self1, W_neigh1, b1, W_self2, W_neigh2, b2):
  ei = edge_index.astype(jnp.int32)

  t1, s1 = _tc_layer1(features, W_neigh1, W_self1, b1.reshape(1, -1))
  p1, degp = _sc_scatter_l1(t1, ei)
  t2, s2, recip = _tc_mid(p1[0], p1[1], degp[0], degp[1], s1,
                          W_neigh2, W_self2, b2.reshape(1, -1))
  (p2,) = _sc_scatter_l2(t2, ei)
  return _tc_final(p2[0], p2[1], s2, recip)
